# baseline (speedup 1.0000x reference)
import functools

import jax
import jax.numpy as jnp
from jax import lax
from jax.experimental import pallas as pl
from jax.experimental.pallas import tpu as pltpu
from jax.experimental.pallas import tpu_sc as plsc

NUM_EMB = 1_000_000
DIM = 64
PAD_DIM = 128
BATCH = 16384
FIELDS = 26

NC = 2
NS = 16
NW = NC * NS
B_PER_W = BATCH // NW
BLK = 128
NBLK = B_PER_W // BLK
NBLOCKS = FIELDS * NBLK
PITCH = BLK + 1

_mesh = plsc.VectorSubcoreMesh(core_axis_name="c", subcore_axis_name="s")


@functools.partial(
    pl.kernel,
    out_type=jax.ShapeDtypeStruct((FIELDS, DIM, BATCH), jnp.float32),
    mesh=_mesh,
    scratch_types=[
        pltpu.VMEM((FIELDS, B_PER_W), jnp.int32),
        pltpu.VMEM((2, BLK, DIM), jnp.float32),
        pltpu.VMEM((DIM, PITCH), jnp.float32),
        pltpu.SemaphoreType.DMA,
        pltpu.SemaphoreType.DMA,
    ],
    compiler_params=pltpu.CompilerParams(
        use_tc_tiling_on_sc=False, needs_layout_passes=False
    ),
)
def _gather_kernel(idxt_hbm, table_hbm, out_hbm, idxt_v, rows_v, trans_v,
                   gsem0, gsem1):
    wid = lax.axis_index("s") * NC + lax.axis_index("c")
    base = wid * B_PER_W
    pltpu.sync_copy(idxt_hbm.at[:, pl.ds(base, B_PER_W)], idxt_v)
    lanes = lax.iota(jnp.int32, 16)

    def fire(g, buf, sem):
        f = g // NBLK
        bb = (g % NBLK) * BLK
        pltpu.async_copy(
            table_hbm.at[idxt_v.at[f, pl.ds(bb, BLK)]], rows_v.at[buf], sem
        )

    def drain(buf, sem):
        pltpu.make_async_copy(
            table_hbm.at[idxt_v.at[0, pl.ds(0, BLK)]], rows_v.at[buf], sem
        ).wait()

    def transpose_store(g, buf):
        f = g // NBLK
        bb = (g % NBLK) * BLK
        for b in range(BLK):
            b_vec = jnp.full((16,), b, jnp.int32)
            for jj in range(DIM // 16):
                vals = rows_v[buf, b, pl.ds(16 * jj, 16)]
                plsc.store_scatter(trans_v, [lanes + (16 * jj), b_vec], vals)
        pltpu.sync_copy(
            trans_v.at[:, pl.ds(0, BLK)],
            out_hbm.at[f, :, pl.ds(base + bb, BLK)],
        )

    fire(0, 0, gsem0)

    def body(k, carry):
        del carry
        e, o = 2 * k, 2 * k + 1
        fire(o, 1, gsem1)
        drain(0, gsem0)
        transpose_store(e, 0)

        @pl.when(k < NBLOCKS // 2 - 1)
        def _():
            fire(o + 1, 0, gsem0)

        drain(1, gsem1)
        transpose_store(o, 1)
        return 0

    lax.fori_loop(0, NBLOCKS // 2, body, 0)


def kernel(token_ids, weights):
    wpad = jnp.pad(weights, ((0, 0), (0, PAD_DIM - DIM)))
    w2 = wpad.reshape(2 * NUM_EMB, DIM)
    idx2t = token_ids.astype(jnp.int32).T * 2
    outt = _gather_kernel(idx2t, w2)
    return outt.transpose(2, 0, 1)


# speedup vs baseline: 1.0010x; 1.0010x over previous
import functools

import jax
import jax.numpy as jnp
from jax import lax
from jax.experimental import pallas as pl
from jax.experimental.pallas import tpu as pltpu
from jax.experimental.pallas import tpu_sc as plsc

NUM_EMB = 1_000_000
DIM = 64
PAD_DIM = 128
BATCH = 16384
FIELDS = 26

NC = 2
NS = 16
NW = NC * NS
B_PER_W = BATCH // NW
BLK = 128
NBLK = B_PER_W // BLK
NBLOCKS = FIELDS * NBLK
PITCH = BLK + 1

_mesh = plsc.VectorSubcoreMesh(core_axis_name="c", subcore_axis_name="s")


@functools.partial(
    pl.kernel,
    out_type=jax.ShapeDtypeStruct((FIELDS, DIM, BATCH), jnp.float32),
    mesh=_mesh,
    scratch_types=[
        pltpu.VMEM((FIELDS, B_PER_W), jnp.int32),
        pltpu.VMEM((2, BLK, DIM), jnp.float32),
        pltpu.VMEM((2, DIM, PITCH), jnp.float32),
        pltpu.SemaphoreType.DMA,
        pltpu.SemaphoreType.DMA,
        pltpu.SemaphoreType.DMA,
        pltpu.SemaphoreType.DMA,
    ],
    compiler_params=pltpu.CompilerParams(
        use_tc_tiling_on_sc=False, needs_layout_passes=False
    ),
)
def _gather_kernel(idxt_hbm, table_hbm, out_hbm, idxt_v, rows_v, trans_v,
                   gsem0, gsem1, ssem0, ssem1):
    wid = lax.axis_index("s") * NC + lax.axis_index("c")
    base = wid * B_PER_W
    pltpu.sync_copy(idxt_hbm.at[:, pl.ds(base, B_PER_W)], idxt_v)
    lanes = lax.iota(jnp.int32, 16)
    ssems = (ssem0, ssem1)

    def fire(g, buf, sem):
        f = g // NBLK
        bb = (g % NBLK) * BLK
        pltpu.async_copy(
            table_hbm.at[idxt_v.at[f, pl.ds(bb, BLK)]], rows_v.at[buf], sem
        )

    def drain(buf, sem):
        pltpu.make_async_copy(
            table_hbm.at[idxt_v.at[0, pl.ds(0, BLK)]], rows_v.at[buf], sem
        ).wait()

    def transpose(buf):
        for b in range(BLK):
            b_vec = jnp.full((16,), b, jnp.int32)
            for jj in range(DIM // 16):
                vals = rows_v[buf, b, pl.ds(16 * jj, 16)]
                plsc.store_scatter(
                    trans_v.at[buf], [lanes + (16 * jj), b_vec], vals
                )

    def fire_store(g, buf):
        f = g // NBLK
        bb = (g % NBLK) * BLK
        pltpu.async_copy(
            trans_v.at[buf, :, pl.ds(0, BLK)],
            out_hbm.at[f, :, pl.ds(base + bb, BLK)],
            ssems[buf],
        )

    def drain_store(buf):
        pltpu.make_async_copy(
            trans_v.at[buf, :, pl.ds(0, BLK)],
            out_hbm.at[0, :, pl.ds(base, BLK)],
            ssems[buf],
        ).wait()

    fire(0, 0, gsem0)

    def body(k, carry):
        del carry
        e, o = 2 * k, 2 * k + 1
        fire(o, 1, gsem1)
        drain(0, gsem0)

        @pl.when(k > 0)
        def _():
            drain_store(0)

        transpose(0)
        fire_store(e, 0)

        @pl.when(k < NBLOCKS // 2 - 1)
        def _():
            fire(o + 1, 0, gsem0)

        drain(1, gsem1)

        @pl.when(k > 0)
        def _():
            drain_store(1)

        transpose(1)
        fire_store(o, 1)
        return 0

    lax.fori_loop(0, NBLOCKS // 2, body, 0)
    drain_store(0)
    drain_store(1)


def kernel(token_ids, weights):
    wpad = jnp.pad(weights, ((0, 0), (0, PAD_DIM - DIM)))
    w2 = wpad.reshape(2 * NUM_EMB, DIM)
    idx2t = token_ids.astype(jnp.int32).T * 2
    outt = _gather_kernel(idx2t, w2)
    return outt.transpose(2, 0, 1)


# speedup vs baseline: 1.0341x; 1.0330x over previous
import functools

import jax
import jax.numpy as jnp
from jax import lax
from jax.experimental import pallas as pl
from jax.experimental.pallas import tpu as pltpu
from jax.experimental.pallas import tpu_sc as plsc

NUM_EMB = 1_000_000
DIM = 64
PAD_DIM = 128
BATCH = 16384
FIELDS = 26

NC = 2
NS = 16
NW = NC * NS
B_PER_W = BATCH // NW
CHUNK_B = 16
NCHUNK = B_PER_W // CHUNK_B
NPAIR = NCHUNK // 2

_mesh = plsc.VectorSubcoreMesh(core_axis_name="c", subcore_axis_name="s")


@functools.partial(
    pl.kernel,
    out_type=jax.ShapeDtypeStruct((BATCH, FIELDS, DIM), jnp.float32),
    mesh=_mesh,
    scratch_types=[
        pltpu.VMEM((B_PER_W, FIELDS), jnp.int32),
        pltpu.VMEM((2, CHUNK_B, FIELDS, DIM), jnp.float32),
        pltpu.SemaphoreType.DMA,
        pltpu.SemaphoreType.DMA,
    ],
    compiler_params=pltpu.CompilerParams(
        use_tc_tiling_on_sc=False, needs_layout_passes=False
    ),
)
def _gather_kernel(idx_hbm, table_hbm, out_hbm, idx_v, rows_v, gsem0, gsem1):
    wid = lax.axis_index("s") * NC + lax.axis_index("c")
    base = wid * B_PER_W
    pltpu.sync_copy(idx_hbm.at[pl.ds(base, B_PER_W), :], idx_v)

    def fire(c, buf, sem):
        s = c * CHUNK_B
        for i in range(CHUNK_B):
            pltpu.async_copy(
                table_hbm.at[idx_v.at[s + i, :]], rows_v.at[buf, i], sem
            )

    def drain(buf, sem):
        for i in range(CHUNK_B):
            pltpu.make_async_copy(
                table_hbm.at[idx_v.at[i, :]], rows_v.at[buf, i], sem
            ).wait()

    def store(c, buf):
        pltpu.sync_copy(
            rows_v.at[buf], out_hbm.at[pl.ds(base + c * CHUNK_B, CHUNK_B)]
        )

    fire(0, 0, gsem0)

    def body(k, carry):
        del carry
        e, o = 2 * k, 2 * k + 1
        fire(o, 1, gsem1)
        drain(0, gsem0)
        store(e, 0)

        @pl.when(k < NPAIR - 1)
        def _():
            fire(o + 1, 0, gsem0)

        drain(1, gsem1)
        store(o, 1)
        return 0

    lax.fori_loop(0, NPAIR, body, 0)


def kernel(token_ids, weights):
    wpad = jnp.pad(weights, ((0, 0), (0, PAD_DIM - DIM)))
    w2 = wpad.reshape(2 * NUM_EMB, DIM)
    idx2 = token_ids.astype(jnp.int32) * 2
    return _gather_kernel(idx2, w2)
